# baseline (speedup 1.0000x reference)
import functools

import jax
import jax.numpy as jnp
from jax import lax
from jax.experimental import pallas as pl
from jax.experimental.pallas import tpu as pltpu
from jax.experimental.pallas import tpu_sc as plsc

_VOCAB = 1000000
_D = 32
_BATCH = 16384
_HIST = 50
_NC, _NS = 2, 16
_NW = _NC * _NS
_NBS = 16
_BW = _BATCH // _NBS
_HH = _HIST // 2
_CHUNK = 128
_KPH = _BW // _CHUNK
_NCH = _HH * _KPH
_NBUF = 8
_L = 16

_mesh = plsc.VectorSubcoreMesh(core_axis_name="c", subcore_axis_name="s")


@functools.partial(
    pl.kernel,
    out_type=jax.ShapeDtypeStruct((_HIST, _D, _BATCH), jnp.float32),
    mesh=_mesh,
    scratch_types=[
        pltpu.VMEM((_HH, _BW), jnp.int32),
        pltpu.VMEM((_NBUF * _CHUNK, _D), jnp.float32),
        pltpu.VMEM((2, _D, _BW), jnp.float32),
        pltpu.SemaphoreType.DMA,
        pltpu.SemaphoreType.DMA,
    ],
    compiler_params=pltpu.CompilerParams(
        use_tc_tiling_on_sc=False, needs_layout_passes=False),
)
def _sc_gather(idx_hbm, table_hbm, out_hbm, idx_v, gbuf, tbuf, gsem, ssem):
    wid = lax.axis_index("s") * _NC + lax.axis_index("c")
    b0 = lax.rem(wid, _NBS) * _BW
    h0 = lax.div(wid, _NBS) * _HH
    pltpu.sync_copy(idx_hbm.at[pl.ds(h0, _HH), pl.ds(b0, _BW)], idx_v)

    def gsrc(j):
        h = lax.div(j, _KPH)
        k = lax.rem(j, _KPH)
        return table_hbm.at[idx_v.at[h, pl.ds(k * _CHUNK, _CHUNK)]]

    def gdst(slot):
        return gbuf.at[pl.ds(slot * _CHUNK, _CHUNK)]

    for j in range(_NBUF - 1):
        pltpu.async_copy(gsrc(j), gdst(j), gsem)

    iota = lax.iota(jnp.int32, 16)
    cols = [jnp.full((16,), d, jnp.int32) for d in range(_D)]

    def body(j, carry):
        slot = lax.rem(j, _NBUF)
        h = lax.div(j, _KPH)
        k = lax.rem(j, _KPH)
        tb = lax.rem(h, 2)
        pltpu.make_async_copy(gsrc(j), gdst(slot), gsem).wait()

        @pl.when(jnp.logical_and(k == 0, h >= 2))
        def _drain_store():
            pltpu.make_async_copy(
                tbuf.at[0], out_hbm.at[0, :, pl.ds(b0, _BW)], ssem).wait()

        base = iota + slot * _CHUNK
        for k16 in range(_CHUNK // _L):
            rows = base + k16 * _L
            for d0 in range(0, _D, 8):
                vs = [plsc.load_gather(gbuf, [rows, cols[d0 + i]])
                      for i in range(8)]
                for i in range(8):
                    tbuf[tb, d0 + i, pl.ds(k * _CHUNK + k16 * _L, _L)] = vs[i]

        @pl.when(j + _NBUF - 1 < _NCH)
        def _start_next():
            pltpu.async_copy(
                gsrc(j + _NBUF - 1),
                gdst(lax.rem(j + _NBUF - 1, _NBUF)), gsem)

        @pl.when(k == _KPH - 1)
        def _store_plane():
            pltpu.async_copy(
                tbuf.at[tb], out_hbm.at[h0 + h, :, pl.ds(b0, _BW)], ssem)

        return carry

    lax.fori_loop(0, _NCH, body, 0)

    for _ in range(2):
        pltpu.make_async_copy(
            tbuf.at[0], out_hbm.at[0, :, pl.ds(b0, _BW)], ssem).wait()


def kernel(input_ids, table):
    idx_t = input_ids.astype(jnp.int32).T
    out = _sc_gather(idx_t, table)
    return out.transpose(2, 0, 1)


# speedup vs baseline: 1.4203x; 1.4203x over previous
import functools

import jax
import jax.numpy as jnp
from jax import lax
from jax.experimental import pallas as pl
from jax.experimental.pallas import tpu as pltpu
from jax.experimental.pallas import tpu_sc as plsc

_VOCAB = 1000000
_D = 32
_BATCH = 16384
_HIST = 50
_NC, _NS = 2, 16
_NW = _NC * _NS
_NBS = 16
_BW = _BATCH // _NBS
_HH = _HIST // 2
_CHUNK = 128
_KPH = _BW // _CHUNK
_NCH = _HH * _KPH
_NBUF = 8
_L = 16

_mesh = plsc.VectorSubcoreMesh(core_axis_name="c", subcore_axis_name="s")


@functools.partial(
    pl.kernel,
    out_type=jax.ShapeDtypeStruct((_HIST, _D, _BATCH), jnp.float32),
    mesh=_mesh,
    scratch_types=[
        pltpu.VMEM((_HH, _BW), jnp.int32),
        pltpu.VMEM((_NBUF * _CHUNK, _D), jnp.float32),
        pltpu.VMEM((2, _D, _BW + 1), jnp.float32),
        pltpu.SemaphoreType.DMA,
        pltpu.SemaphoreType.DMA,
    ],
    compiler_params=pltpu.CompilerParams(
        use_tc_tiling_on_sc=False, needs_layout_passes=False),
)
def _sc_gather(idx_hbm, table_hbm, out_hbm, idx_v, gbuf, tbuf, gsem, ssem):
    wid = lax.axis_index("s") * _NC + lax.axis_index("c")
    b0 = lax.rem(wid, _NBS) * _BW
    h0 = lax.div(wid, _NBS) * _HH
    pltpu.sync_copy(idx_hbm.at[pl.ds(h0, _HH), pl.ds(b0, _BW)], idx_v)

    def gsrc(j):
        h = lax.div(j, _KPH)
        k = lax.rem(j, _KPH)
        return table_hbm.at[idx_v.at[h, pl.ds(k * _CHUNK, _CHUNK)]]

    def gdst(slot):
        return gbuf.at[pl.ds(slot * _CHUNK, _CHUNK)]

    for j in range(_NBUF - 1):
        pltpu.async_copy(gsrc(j), gdst(j), gsem)

    iota = lax.iota(jnp.int32, 16)
    iotb = iota + 16

    def body(j, carry):
        slot = lax.rem(j, _NBUF)
        h = lax.div(j, _KPH)
        k = lax.rem(j, _KPH)
        tb = lax.rem(h, 2)
        pltpu.make_async_copy(gsrc(j), gdst(slot), gsem).wait()

        @pl.when(jnp.logical_and(k == 0, h >= 2))
        def _drain_store():
            for d in range(_D):
                pltpu.make_async_copy(
                    tbuf.at[0, d, pl.ds(0, _BW)],
                    out_hbm.at[0, d, pl.ds(b0, _BW)], ssem).wait()

        tplane = tbuf.at[tb]
        cbase = k * _CHUNK
        for r0 in range(0, _CHUNK, 4):
            vs = [(gbuf[slot * _CHUNK + r, pl.ds(0, _L)],
                   gbuf[slot * _CHUNK + r, pl.ds(_L, _L)])
                  for r in range(r0, r0 + 4)]
            for i, r in enumerate(range(r0, r0 + 4)):
                colv = jnp.broadcast_to(cbase + r, (16,)).astype(jnp.int32)
                plsc.store_scatter(tplane, [iota, colv], vs[i][0])
                plsc.store_scatter(tplane, [iotb, colv], vs[i][1])

        @pl.when(j + _NBUF - 1 < _NCH)
        def _start_next():
            pltpu.async_copy(
                gsrc(j + _NBUF - 1),
                gdst(lax.rem(j + _NBUF - 1, _NBUF)), gsem)

        @pl.when(k == _KPH - 1)
        def _store_plane():
            for d in range(_D):
                pltpu.async_copy(
                    tbuf.at[tb, d, pl.ds(0, _BW)],
                    out_hbm.at[h0 + h, d, pl.ds(b0, _BW)], ssem)

        return carry

    lax.fori_loop(0, _NCH, body, 0)

    for _ in range(2 * _D):
        pltpu.make_async_copy(
            tbuf.at[0, 0, pl.ds(0, _BW)],
            out_hbm.at[0, 0, pl.ds(b0, _BW)], ssem).wait()


def kernel(input_ids, table):
    idx_t = input_ids.astype(jnp.int32).T
    out = _sc_gather(idx_t, table)
    return out.transpose(2, 0, 1)
